# baseline (speedup 1.0000x reference)
import functools

import jax
import jax.numpy as jnp
from jax import lax
from jax.experimental import pallas as pl
from jax.experimental.pallas import tpu as pltpu
from jax.experimental.pallas import tpu_sc as plsc

N = 8192
NUM_CLASSES = 13
RADIUS2 = 0.25
KNN = 16
BETA = 0.5
NUM_SEED = 50
N_ITERS = 5
RBLK = 128
BIG = 1e30


def _msp_kernel(logits_ref, msp_ref, stop_ref):
    l = logits_ref[...]
    m = jnp.max(l, axis=1, keepdims=True)
    s = jnp.sum(jnp.exp(l - m), axis=1, keepdims=True)
    msp = 1.0 / s[:, 0]
    msp_ref[...] = msp
    mean = jnp.sum(msp) / N
    var = jnp.sum((msp - mean) * (msp - mean)) / N
    stop = mean - BETA * jnp.sqrt(jnp.maximum(var, 0.0))
    stop_ref[...] = jnp.full((1, 128), stop, jnp.float32)


def _msp_call(seg_logits):
    return pl.pallas_call(
        _msp_kernel,
        out_shape=(
            jax.ShapeDtypeStruct((N,), jnp.float32),
            jax.ShapeDtypeStruct((1, 128), jnp.float32),
        ),
    )(seg_logits)


def _ball_kernel(cblk_ref, ct_ref, mspf_ref, mspb_ref, dice_ref,
                 bm_ref, summ_ref, mask0_ref):
    i = pl.program_id(0)
    cblk = cblk_ref[...]
    ct = ct_ref[...]
    sqf = jnp.sum(ct * ct, axis=0)[None, :]
    sqb = jnp.sum(cblk * cblk, axis=1)[:, None]
    d2 = sqb + sqf - 2.0 * jnp.dot(cblk, ct, preferred_element_type=jnp.float32)
    jl = lax.broadcasted_iota(jnp.int32, (RBLK, N), 1)
    valid = (d2 <= RADIUS2).astype(jnp.int32)
    bits = valid << (jl & 31)
    words = jnp.sum(bits.reshape(RBLK, N // 32, 32), axis=2)
    bm_ref[...] = words.reshape(RBLK * (N // 32))
    jw = lax.broadcasted_iota(jnp.int32, (RBLK, N // 32), 1)
    wnz = (words != 0).astype(jnp.int32) << (jw & 31)
    summ_ref[...] = jnp.sum(wnz.reshape(RBLK, N // 1024, 32),
                            axis=2).reshape(RBLK * (N // 1024))

    mspf = mspf_ref[0, :][None, :]
    mspb = mspb_ref[...][:, None]
    gi = i * RBLK + lax.broadcasted_iota(jnp.int32, (RBLK, 1), 0)
    rank = (jnp.sum((mspf < mspb).astype(jnp.int32), axis=1)
            + jnp.sum(((mspf == mspb) & (jl < gi)).astype(jnp.int32), axis=1))
    dice = dice_ref[...]
    hit = jnp.any(rank[:, None] == dice, axis=1)
    mask0_ref[...] = hit.astype(jnp.int32)


def _ball_call(coord, msp, dice2):
    coordT = coord.T
    msp2 = msp.reshape(1, N)
    grid = (N // RBLK,)
    return pl.pallas_call(
        _ball_kernel,
        grid=grid,
        in_specs=[
            pl.BlockSpec((RBLK, 3), lambda i: (i, 0)),
            pl.BlockSpec((3, N), lambda i: (0, 0)),
            pl.BlockSpec((1, N), lambda i: (0, 0)),
            pl.BlockSpec((RBLK,), lambda i: (i,)),
            pl.BlockSpec((1, 64), lambda i: (0, 0)),
        ],
        out_specs=(
            pl.BlockSpec((RBLK * (N // 32),), lambda i: (i,)),
            pl.BlockSpec((RBLK * (N // 1024),), lambda i: (i,)),
            pl.BlockSpec((RBLK,), lambda i: (i,)),
        ),
        out_shape=(
            jax.ShapeDtypeStruct((N * (N // 32),), jnp.int32),
            jax.ShapeDtypeStruct((N * (N // 1024),), jnp.int32),
            jax.ShapeDtypeStruct((N,), jnp.int32),
        ),
    )(coord, coordT, msp2, msp, dice2)


NG = N // 16


def _sdiv(a, b):
    return (jnp.full((16,), a, jnp.float32) / jnp.full((16,), b, jnp.float32))[0]


def _vsqrt(a):
    a = jnp.maximum(a, 1e-30)
    i = plsc.bitcast(a, jnp.int32)
    y = plsc.bitcast(0x5F3759DF - (i >> 1), jnp.float32)
    for _ in range(3):
        y = y * (1.5 - 0.5 * a * y * y)
    return a * y


def _sc_grow_body(x_h, y_h, z_h, s_h, bm_h, summ_h, mask0_h, params_h,
                  out_h, nbr_h,
                  x_v, y_v, z_v, s_v, d_v, sim_v, csim_v,
                  mask_v, cand_v, idxl_v, buf_v, sq_v, bm_v, summ_v, sem):
    cid = lax.axis_index("c")
    sid = lax.axis_index("s")
    tile0 = (cid == 0) & (sid == 0)

    @pl.when(cid == 0)
    def _extract():
        lane = lax.iota(jnp.int32, 16)
        zeros_i = jnp.zeros((16,), jnp.int32)
        neg1_v = jnp.full((16,), -1, jnp.int32)
        big_v = jnp.full((16,), BIG, jnp.float32)
        lane0 = lane == 0
        pltpu.sync_copy(x_h, x_v.at[pl.ds(0, N)])
        pltpu.sync_copy(y_h, y_v.at[pl.ds(0, N)])
        pltpu.sync_copy(z_h, z_v.at[pl.ds(0, N)])

        def sqb(g, _):
            sl = pl.ds(g * 16, 16)
            sq_v[sl] = (x_v[sl] * x_v[sl] + y_v[sl] * y_v[sl]
                        + z_v[sl] * z_v[sl])
            return 0

        lax.fori_loop(0, NG, sqb, 0, unroll=4)

        def _ctz(w):
            low = w & (-w)
            f = lax.convert_element_type(low, jnp.float32)
            b = lax.bitcast_convert_type(f, jnp.int32)
            return ((b >> 23) & 255) - 127

        base = sid * (N // 16)
        for chunk in range(N // 16 // 64):
            rb = base + chunk * 64
            pltpu.sync_copy(bm_h.at[pl.ds(rb * (N // 32), 64 * (N // 32))],
                            bm_v.at[pl.ds(0, 64 * (N // 32))])
            pltpu.sync_copy(summ_h.at[pl.ds(rb * (N // 1024),
                                            64 * (N // 1024))],
                            summ_v.at[pl.ds(0, 64 * (N // 1024))])

            def pf(r, _):
                for k in range(8):
                    buf_v[r, pl.ds(k * 16, 16)] = neg1_v
                return 0

            lax.fori_loop(0, 64, pf, 0)

            def row_body(r2, _):
                gi = rb + r2

                def wloop(w, cur):
                    sw0 = summ_v[pl.ds(r2 * (N // 1024) + w, 16)][0]

                    def swb(c):
                        sw, cu = c
                        widx = w * 32 + _ctz(sw)
                        wv0 = bm_v[pl.ds(r2 * (N // 32) + widx, 16)][0]

                        def wvb(c2):
                            wv, cu2 = c2
                            pidx = widx * 32 + _ctz(wv)
                            plsc.store_scatter(
                                cand_v, [jnp.full((16,), cu2, jnp.int32)],
                                jnp.full((16,), pidx, jnp.int32), mask=lane0)
                            return (wv & (wv - 1), cu2 + 1)

                        _, cu3 = lax.while_loop(lambda c2: c2[0] != 0, wvb,
                                                (wv0, cu))
                        return (sw & (sw - 1), cu3)

                    _, curf = lax.while_loop(lambda c: c[0] != 0, swb,
                                             (sw0, cur))
                    return curf

                nc = lax.fori_loop(0, N // 1024, wloop, jnp.int32(0))
                cand_v[pl.ds(nc, 16)] = zeros_i
                xi = x_v[pl.ds(gi, 16)][0]
                yi = y_v[pl.ds(gi, 16)][0]
                zi = z_v[pl.ds(gi, 16)][0]
                sqi = sq_v[pl.ds(gi, 16)][0]
                nch = (nc + 15) >> 4

                def dch(c, _):
                    sl = pl.ds(c * 16, 16)
                    iv = cand_v[sl]
                    xg = plsc.load_gather(x_v, [iv])
                    yg = plsc.load_gather(y_v, [iv])
                    zg = plsc.load_gather(z_v, [iv])
                    sqg = plsc.load_gather(sq_v, [iv])
                    d2c = sqi + sqg - 2.0 * (xi * xg + yi * yg + zi * zg)
                    pos = c * 16 + lane
                    d_v[sl] = jnp.where(pos < nc, d2c, BIG)
                    return 0

                lax.fori_loop(0, nch, dch, 0)

                def light():
                    iv = cand_v[pl.ds(0, 16)]
                    buf_v[r2, pl.ds(0, 16)] = jnp.where(lane < nc, iv, -1)

                def heavy():
                    def kext(k, _):
                        def ms(c, acc):
                            return jnp.minimum(acc, d_v[pl.ds(c * 16, 16)])

                        m = jnp.min(lax.fori_loop(0, nch, ms, big_v))

                        def isc(c, acc):
                            sl = pl.ds(c * 16, 16)
                            return jnp.minimum(acc, jnp.min(jnp.where(
                                d_v[sl] == m, cand_v[sl], N)))

                        mi = lax.fori_loop(0, nch, isc, jnp.int32(N))

                        def ko(c, _):
                            sl = pl.ds(c * 16, 16)
                            hit = (d_v[sl] == m) & (cand_v[sl] == mi)
                            d_v[sl] = jnp.where(hit, BIG, d_v[sl])
                            return 0

                        lax.fori_loop(0, nch, ko, 0)
                        plsc.store_scatter(
                            buf_v,
                            [jnp.full((16,), r2, jnp.int32),
                             jnp.full((16,), k, jnp.int32)],
                            jnp.full((16,), mi, jnp.int32), mask=lane0)
                        return 0

                    lax.fori_loop(0, KNN, kext, 0)

                lax.cond(nc <= KNN, light, heavy)
                return 0

            lax.fori_loop(0, 64, row_body, 0)
            pltpu.sync_copy(buf_v.at[pl.ds(0, 64), :],
                            nbr_h.at[pl.ds(rb, 64), :])

    plsc.subcore_barrier()

    @pl.when(tile0)
    def _():
        pltpu.sync_copy(s_h, s_v)
        pltpu.sync_copy(mask0_h, mask_v)
        pltpu.sync_copy(params_h, csim_v.at[pl.ds(0, 16)])
        stop_cond = csim_v[pl.ds(0, 16)][0]
        zeros_i = jnp.zeros((16,), jnp.int32)
        zeros_f = jnp.zeros((16,), jnp.float32)
        ones_i = jnp.ones((16,), jnp.int32)
        lane = lax.iota(jnp.int32, 16)

        def iteration(_, carry):
            def red_body(g, acc):
                cnt_a, sx_a, sy_a, sz_a, ss_a = acc
                mv = mask_v[pl.ds(g * 16, 16)] > 0
                f = jnp.where(mv, 1.0, 0.0)
                return (cnt_a + f,
                        sx_a + jnp.where(mv, x_v[pl.ds(g * 16, 16)], 0.0),
                        sy_a + jnp.where(mv, y_v[pl.ds(g * 16, 16)], 0.0),
                        sz_a + jnp.where(mv, z_v[pl.ds(g * 16, 16)], 0.0),
                        ss_a + jnp.where(mv, s_v[pl.ds(g * 16, 16)], 0.0))

            cnt_a, sx_a, sy_a, sz_a, ss_a = lax.fori_loop(
                0, NG, red_body, (zeros_f, zeros_f, zeros_f, zeros_f, zeros_f),
                unroll=4)
            cnt = jnp.sum(cnt_a)
            denom = jnp.maximum(cnt, 1.0)
            rden = _sdiv(1.0, denom)
            mx = jnp.sum(sx_a) * rden
            my = jnp.sum(sy_a) * rden
            mz = jnp.sum(sz_a) * rden
            ms = jnp.sum(ss_a) * rden
            cond_met = ((ms > stop_cond) & (cnt > 0.01 * N) & (cnt > 50.0))

            @pl.when(jnp.logical_not(cond_met))
            def _grow():
                def cz_body(g, nm):
                    cand_v[pl.ds(g * 16, 16)] = zeros_i
                    mv = mask_v[pl.ds(g * 16, 16)] > 0
                    mi = jnp.where(mv, 1, 0)
                    pos = nm + plsc.cumsum(mi) - 1
                    plsc.store_scatter(idxl_v, [pos], g * 16 + lane, mask=mv)
                    return nm + jnp.sum(mi)

                nm = lax.fori_loop(0, NG, cz_body, jnp.int32(0), unroll=4)
                for t in range(8):
                    idxl_v[pl.ds(nm + t * 16, 16)] = zeros_i

                def chunk_body(c, _):
                    pltpu.async_copy(
                        nbr_h.at[idxl_v.at[pl.ds(c * 128, 128)]], buf_v,
                        sem).wait()

                    def row_body(r, __):
                        tgt = buf_v[r, pl.ds(0, 16)]
                        valid = tgt >= 0
                        plsc.store_scatter(cand_v,
                                           [jnp.where(valid, tgt, 0)],
                                           ones_i, mask=valid)
                        return 0

                    rhi = jnp.minimum(128, nm - c * 128)
                    lax.fori_loop(0, rhi, row_body, 0)
                    return 0

                nchunks = (nm + 127) >> 7
                lax.fori_loop(0, nchunks, chunk_body, 0)

                def dm_body(g, acc):
                    dmin_a, dmax_a = acc
                    sl = pl.ds(g * 16, 16)
                    cv = (cand_v[sl] > 0) & (mask_v[sl] == 0)
                    cand_v[sl] = jnp.where(cv, 1, 0)
                    dx = x_v[sl] - mx
                    dy = y_v[sl] - my
                    dz = z_v[sl] - mz
                    d = _vsqrt(dx * dx + dy * dy + dz * dz)
                    d_v[sl] = d
                    return (jnp.minimum(dmin_a, jnp.where(cv, d, BIG)),
                            jnp.maximum(dmax_a, jnp.where(cv, d, -BIG)))

                dmin_a, dmax_a = lax.fori_loop(
                    0, NG, dm_body,
                    (jnp.full((16,), BIG, jnp.float32),
                     jnp.full((16,), -BIG, jnp.float32)), unroll=4)
                dmin = jnp.min(dmin_a)
                dmax = jnp.max(dmax_a)

                inv_rng = _sdiv(1.0, dmax - dmin + 0.001)

                def sim_body(g, nc):
                    sl = pl.ds(g * 16, 16)
                    cv = cand_v[sl] > 0
                    dist_sim = 1.0 - (d_v[sl] - dmin) * inv_rng
                    conf_sim = jnp.exp(-jnp.abs(s_v[sl] - ms))
                    sim = 0.4 * dist_sim + 0.6 * conf_sim
                    sim = jnp.where(cv, sim, 0.0)
                    sim_v[sl] = sim
                    ci = jnp.where(cv, 1, 0)
                    pos = nc + plsc.cumsum(ci) - 1
                    plsc.store_scatter(csim_v, [pos], sim, mask=cv)
                    return nc + jnp.sum(ci)

                nc = lax.fori_loop(0, NG, sim_body, jnp.int32(0), unroll=4)
                csim_v[pl.ds(nc, 16)] = zeros_f

                k_sel = (0.4 * nc.astype(jnp.float32)).astype(jnp.int32)
                ncg = (nc + 15) >> 4

                def bs_body(_, lohi):
                    lo, hi = lohi
                    mid = (lo + hi) >> 1
                    midf = lax.bitcast_convert_type(mid, jnp.float32)

                    def cb(g, a):
                        return a + jnp.where(csim_v[pl.ds(g * 16, 16)] >= midf,
                                             1, 0)

                    c = jnp.sum(lax.fori_loop(0, ncg, cb, zeros_i))
                    take = c >= k_sel
                    return (jnp.where(take, mid, lo), jnp.where(take, hi, mid))

                lo, _hi = lax.fori_loop(0, 31, bs_body,
                                        (jnp.int32(0), jnp.int32(0x3F800001)))
                thr = lax.bitcast_convert_type(lo, jnp.float32)
                do_sel = k_sel > 0

                def sel_body(g, _):
                    sl = pl.ds(g * 16, 16)
                    sel = (cand_v[sl] > 0) & (sim_v[sl] >= thr) & do_sel
                    mask_v[sl] = jnp.where(sel, 1, mask_v[sl])
                    return 0

                lax.fori_loop(0, NG, sel_body, 0, unroll=4)

            return carry

        lax.fori_loop(0, N_ITERS, iteration, 0)
        pltpu.sync_copy(mask_v, out_h)


def _sc_grow_call(xs, ys, zs, ss, bm, summ, mask0, params):
    mesh = plsc.VectorSubcoreMesh(core_axis_name="c", subcore_axis_name="s",
                                  num_cores=2, num_subcores=16)
    f32 = jnp.float32
    i32 = jnp.int32
    kern = pl.kernel(
        _sc_grow_body,
        out_type=(
            jax.ShapeDtypeStruct((N,), i32),
            jax.ShapeDtypeStruct((N, 128), i32),
        ),
        mesh=mesh,
        scratch_types=[
            pltpu.VMEM((N + 16,), f32),
            pltpu.VMEM((N + 16,), f32),
            pltpu.VMEM((N + 16,), f32),
            pltpu.VMEM((N,), f32),
            pltpu.VMEM((N + 16,), f32),
            pltpu.VMEM((N,), f32),
            pltpu.VMEM((N + 16,), f32),
            pltpu.VMEM((N,), i32),
            pltpu.VMEM((N + 16,), i32),
            pltpu.VMEM((N + 128,), i32),
            pltpu.VMEM((128, 128), i32),
            pltpu.VMEM((N + 16,), f32),
            pltpu.VMEM((64 * (N // 32) + 16,), i32),
            pltpu.VMEM((64 * (N // 1024) + 16,), i32),
            pltpu.SemaphoreType.DMA,
        ],
        compiler_params=pltpu.CompilerParams(needs_layout_passes=False),
    )
    pseudo, _nbr = kern(xs, ys, zs, ss, bm, summ, mask0, params)
    return pseudo


def _final_kernel(lf_ref, seg_ref, pseudo_ref, score_ref, loss_ref):
    lf = lf_ref[...]
    m = jnp.max(lf, axis=1, keepdims=True)
    ex = jnp.exp(lf - m)
    s = jnp.sum(ex, axis=1, keepdims=True)
    logp = lf - m - jnp.log(s)
    tgt = jnp.where(pseudo_ref[...] > 0, NUM_CLASSES, seg_ref[...])
    cols = lax.broadcasted_iota(jnp.int32, (N, NUM_CLASSES + 1), 1)
    nll = -jnp.sum(jnp.where(cols == tgt[:, None], logp, 0.0), axis=1)
    loss_ref[...] = jnp.full((1, 128), jnp.sum(nll) / N, jnp.float32)
    score_ref[...] = ex[:, NUM_CLASSES] / s[:, 0]


def _final_call(logits_full, segment, pseudo):
    return pl.pallas_call(
        _final_kernel,
        out_shape=(
            jax.ShapeDtypeStruct((N,), jnp.float32),
            jax.ShapeDtypeStruct((1, 128), jnp.float32),
        ),
    )(logits_full, segment, pseudo)


def kernel(coord, seg_logits, score, segment):
    msp, stop2 = _msp_call(seg_logits)
    dice = jax.random.randint(jax.random.key(1), (NUM_SEED,), 0,
                              int(0.2 * N)).astype(jnp.int32)
    dice2 = jnp.concatenate(
        [dice, jnp.full((64 - NUM_SEED,), -1, jnp.int32)]).reshape(1, 64)
    bm, summ, mask0 = _ball_call(coord, msp, dice2)
    params = jnp.zeros((16,), jnp.float32).at[0].set(stop2[0, 0])
    xs = jnp.asarray(coord[:, 0])
    ys = jnp.asarray(coord[:, 1])
    zs = jnp.asarray(coord[:, 2])
    pseudo = _sc_grow_call(xs, ys, zs, msp, bm, summ, mask0, params)
    logits_full = jnp.concatenate([seg_logits, score], axis=-1)
    score_out, loss2 = _final_call(logits_full, segment.astype(jnp.int32),
                                   pseudo)
    return (score_out, loss2[0, 0])


# speedup vs baseline: 1.5324x; 1.5324x over previous
import functools

import jax
import jax.numpy as jnp
from jax import lax
from jax.experimental import pallas as pl
from jax.experimental.pallas import tpu as pltpu
from jax.experimental.pallas import tpu_sc as plsc

N = 8192
NUM_CLASSES = 13
RADIUS2 = 0.25
KNN = 16
BETA = 0.5
NUM_SEED = 50
N_ITERS = 5
RBLK = 256
BIG = 1e30


def _msp_kernel(logits_ref, msp_ref, stop_ref):
    l = logits_ref[...]
    m = jnp.max(l, axis=1, keepdims=True)
    s = jnp.sum(jnp.exp(l - m), axis=1, keepdims=True)
    msp = 1.0 / s[:, 0]
    msp_ref[...] = msp
    mean = jnp.sum(msp) / N
    var = jnp.sum((msp - mean) * (msp - mean)) / N
    stop = mean - BETA * jnp.sqrt(jnp.maximum(var, 0.0))
    stop_ref[...] = jnp.full((1, 128), stop, jnp.float32)


def _msp_call(seg_logits):
    return pl.pallas_call(
        _msp_kernel,
        out_shape=(
            jax.ShapeDtypeStruct((N,), jnp.float32),
            jax.ShapeDtypeStruct((1, 128), jnp.float32),
        ),
    )(seg_logits)


def _ball_kernel(cblk_ref, ct_ref, mspf_ref, mspb_ref, dice_ref,
                 nbr_ref, mask0_ref):
    i = pl.program_id(0)
    cblk = cblk_ref[...]
    ct = ct_ref[...]
    sqf = jnp.sum(ct * ct, axis=0)[None, :]
    sqb = jnp.sum(cblk * cblk, axis=1)[:, None]
    d2 = sqb + sqf - 2.0 * jnp.dot(cblk, ct, preferred_element_type=jnp.float32)
    jl = lax.broadcasted_iota(jnp.int32, (RBLK, N), 1)
    d2v = jnp.where(d2 <= RADIUS2, d2, BIG)
    cols = []
    for _ in range(KNN):
        m = jnp.min(d2v, axis=1, keepdims=True)
        idx = jnp.min(jnp.where(d2v == m, jl, N), axis=1, keepdims=True)
        cols.append(jnp.where(m[:, 0] < 0.5 * BIG, idx[:, 0], -1))
        d2v = jnp.where(jl == idx, BIG, d2v)
    nbr_blk = jnp.stack(cols, axis=1).astype(jnp.int32)
    nbr_ref[...] = jnp.concatenate(
        [nbr_blk, jnp.full((RBLK, 128 - KNN), -1, jnp.int32)], axis=1)

    mspf = mspf_ref[0, :][None, :]
    mspb = mspb_ref[...][:, None]
    gi = i * RBLK + lax.broadcasted_iota(jnp.int32, (RBLK, 1), 0)
    rank = (jnp.sum((mspf < mspb).astype(jnp.int32), axis=1)
            + jnp.sum(((mspf == mspb) & (jl < gi)).astype(jnp.int32), axis=1))
    dice = dice_ref[...]
    hit = jnp.any(rank[:, None] == dice, axis=1)
    mask0_ref[...] = hit.astype(jnp.int32)


def _ball_call(coord, msp, dice2):
    coordT = coord.T
    msp2 = msp.reshape(1, N)
    grid = (N // RBLK,)
    return pl.pallas_call(
        _ball_kernel,
        grid=grid,
        in_specs=[
            pl.BlockSpec((RBLK, 3), lambda i: (i, 0)),
            pl.BlockSpec((3, N), lambda i: (0, 0)),
            pl.BlockSpec((1, N), lambda i: (0, 0)),
            pl.BlockSpec((RBLK,), lambda i: (i,)),
            pl.BlockSpec((1, 64), lambda i: (0, 0)),
        ],
        out_specs=(
            pl.BlockSpec((RBLK, 128), lambda i: (i, 0)),
            pl.BlockSpec((RBLK,), lambda i: (i,)),
        ),
        out_shape=(
            jax.ShapeDtypeStruct((N, 128), jnp.int32),
            jax.ShapeDtypeStruct((N,), jnp.int32),
        ),
    )(coord, coordT, msp2, msp, dice2)


NG = N // 16


def _sdiv(a, b):
    return (jnp.full((16,), a, jnp.float32) / jnp.full((16,), b, jnp.float32))[0]


def _vsqrt(a):
    a = jnp.maximum(a, 1e-30)
    i = plsc.bitcast(a, jnp.int32)
    y = plsc.bitcast(0x5F3759DF - (i >> 1), jnp.float32)
    for _ in range(3):
        y = y * (1.5 - 0.5 * a * y * y)
    return a * y


def _sc_grow_body(x_h, y_h, z_h, s_h, nbr_h, mask0_h, params_h, out_h,
                  x_v, y_v, z_v, s_v, d_v, sim_v, csim_v,
                  mask_v, cand_v, idxl_v, buf_v, sem):
    tile0 = (lax.axis_index("c") == 0) & (lax.axis_index("s") == 0)

    @pl.when(tile0)
    def _():
        pltpu.sync_copy(x_h, x_v)
        pltpu.sync_copy(y_h, y_v)
        pltpu.sync_copy(z_h, z_v)
        pltpu.sync_copy(s_h, s_v)
        pltpu.sync_copy(mask0_h, mask_v)
        pltpu.sync_copy(params_h, csim_v.at[pl.ds(0, 16)])
        stop_cond = csim_v[pl.ds(0, 16)][0]
        zeros_i = jnp.zeros((16,), jnp.int32)
        zeros_f = jnp.zeros((16,), jnp.float32)
        ones_i = jnp.ones((16,), jnp.int32)
        lane = lax.iota(jnp.int32, 16)

        def iteration(_, carry):
            def red_body(g, acc):
                cnt_a, sx_a, sy_a, sz_a, ss_a = acc
                mv = mask_v[pl.ds(g * 16, 16)] > 0
                f = jnp.where(mv, 1.0, 0.0)
                return (cnt_a + f,
                        sx_a + jnp.where(mv, x_v[pl.ds(g * 16, 16)], 0.0),
                        sy_a + jnp.where(mv, y_v[pl.ds(g * 16, 16)], 0.0),
                        sz_a + jnp.where(mv, z_v[pl.ds(g * 16, 16)], 0.0),
                        ss_a + jnp.where(mv, s_v[pl.ds(g * 16, 16)], 0.0))

            cnt_a, sx_a, sy_a, sz_a, ss_a = lax.fori_loop(
                0, NG, red_body, (zeros_f, zeros_f, zeros_f, zeros_f, zeros_f),
                unroll=4)
            cnt = jnp.sum(cnt_a)
            denom = jnp.maximum(cnt, 1.0)
            rden = _sdiv(1.0, denom)
            mx = jnp.sum(sx_a) * rden
            my = jnp.sum(sy_a) * rden
            mz = jnp.sum(sz_a) * rden
            ms = jnp.sum(ss_a) * rden
            cond_met = ((ms > stop_cond) & (cnt > 0.01 * N) & (cnt > 50.0))

            @pl.when(jnp.logical_not(cond_met))
            def _grow():
                def cz_body(g, nm):
                    cand_v[pl.ds(g * 16, 16)] = zeros_i
                    mv = mask_v[pl.ds(g * 16, 16)] > 0
                    mi = jnp.where(mv, 1, 0)
                    pos = nm + plsc.cumsum(mi) - 1
                    plsc.store_scatter(idxl_v, [pos], g * 16 + lane, mask=mv)
                    return nm + jnp.sum(mi)

                nm = lax.fori_loop(0, NG, cz_body, jnp.int32(0), unroll=4)
                for t in range(8):
                    idxl_v[pl.ds(nm + t * 16, 16)] = zeros_i

                def chunk_body(c, _):
                    pltpu.async_copy(
                        nbr_h.at[idxl_v.at[pl.ds(c * 128, 128)]], buf_v,
                        sem).wait()

                    def row_body(r, __):
                        tgt = buf_v[r, pl.ds(0, 16)]
                        valid = tgt >= 0
                        plsc.store_scatter(cand_v,
                                           [jnp.where(valid, tgt, 0)],
                                           ones_i, mask=valid)
                        return 0

                    rhi = jnp.minimum(128, nm - c * 128)
                    lax.fori_loop(0, rhi, row_body, 0)
                    return 0

                nchunks = (nm + 127) >> 7
                lax.fori_loop(0, nchunks, chunk_body, 0)

                def dm_body(g, acc):
                    dmin_a, dmax_a = acc
                    sl = pl.ds(g * 16, 16)
                    cv = (cand_v[sl] > 0) & (mask_v[sl] == 0)
                    cand_v[sl] = jnp.where(cv, 1, 0)
                    dx = x_v[sl] - mx
                    dy = y_v[sl] - my
                    dz = z_v[sl] - mz
                    d = _vsqrt(dx * dx + dy * dy + dz * dz)
                    d_v[sl] = d
                    return (jnp.minimum(dmin_a, jnp.where(cv, d, BIG)),
                            jnp.maximum(dmax_a, jnp.where(cv, d, -BIG)))

                dmin_a, dmax_a = lax.fori_loop(
                    0, NG, dm_body,
                    (jnp.full((16,), BIG, jnp.float32),
                     jnp.full((16,), -BIG, jnp.float32)), unroll=4)
                dmin = jnp.min(dmin_a)
                dmax = jnp.max(dmax_a)

                inv_rng = _sdiv(1.0, dmax - dmin + 0.001)

                def sim_body(g, nc):
                    sl = pl.ds(g * 16, 16)
                    cv = cand_v[sl] > 0
                    dist_sim = 1.0 - (d_v[sl] - dmin) * inv_rng
                    conf_sim = jnp.exp(-jnp.abs(s_v[sl] - ms))
                    sim = 0.4 * dist_sim + 0.6 * conf_sim
                    sim = jnp.where(cv, sim, 0.0)
                    sim_v[sl] = sim
                    ci = jnp.where(cv, 1, 0)
                    pos = nc + plsc.cumsum(ci) - 1
                    plsc.store_scatter(csim_v, [pos], sim, mask=cv)
                    return nc + jnp.sum(ci)

                nc = lax.fori_loop(0, NG, sim_body, jnp.int32(0), unroll=4)
                csim_v[pl.ds(nc, 16)] = zeros_f

                k_sel = (0.4 * nc.astype(jnp.float32)).astype(jnp.int32)
                ncg = (nc + 15) >> 4

                def bs_body(_, lohi):
                    lo, hi = lohi
                    mid = (lo + hi) >> 1
                    midf = lax.bitcast_convert_type(mid, jnp.float32)

                    def cb(g, a):
                        return a + jnp.where(csim_v[pl.ds(g * 16, 16)] >= midf,
                                             1, 0)

                    c = jnp.sum(lax.fori_loop(0, ncg, cb, zeros_i))
                    take = c >= k_sel
                    return (jnp.where(take, mid, lo), jnp.where(take, hi, mid))

                lo, _hi = lax.fori_loop(0, 31, bs_body,
                                        (jnp.int32(0), jnp.int32(0x3F800001)))
                thr = lax.bitcast_convert_type(lo, jnp.float32)
                do_sel = k_sel > 0

                def sel_body(g, _):
                    sl = pl.ds(g * 16, 16)
                    sel = (cand_v[sl] > 0) & (sim_v[sl] >= thr) & do_sel
                    mask_v[sl] = jnp.where(sel, 1, mask_v[sl])
                    return 0

                lax.fori_loop(0, NG, sel_body, 0, unroll=4)

            return carry

        lax.fori_loop(0, N_ITERS, iteration, 0)
        pltpu.sync_copy(mask_v, out_h)


def _sc_grow_call(xs, ys, zs, ss, nbr, mask0, params):
    mesh = plsc.VectorSubcoreMesh(core_axis_name="c", subcore_axis_name="s",
                                  num_cores=2, num_subcores=16)
    f32 = jnp.float32
    i32 = jnp.int32
    kern = pl.kernel(
        _sc_grow_body,
        out_type=jax.ShapeDtypeStruct((N,), i32),
        mesh=mesh,
        scratch_types=[
            pltpu.VMEM((N,), f32),
            pltpu.VMEM((N,), f32),
            pltpu.VMEM((N,), f32),
            pltpu.VMEM((N,), f32),
            pltpu.VMEM((N,), f32),
            pltpu.VMEM((N,), f32),
            pltpu.VMEM((N + 16,), f32),
            pltpu.VMEM((N,), i32),
            pltpu.VMEM((N,), i32),
            pltpu.VMEM((N + 128,), i32),
            pltpu.VMEM((128, 128), i32),
            pltpu.SemaphoreType.DMA,
        ],
        compiler_params=pltpu.CompilerParams(needs_layout_passes=False),
    )
    return kern(xs, ys, zs, ss, nbr, mask0, params)


def _final_kernel(lf_ref, seg_ref, pseudo_ref, score_ref, loss_ref):
    lf = lf_ref[...]
    m = jnp.max(lf, axis=1, keepdims=True)
    ex = jnp.exp(lf - m)
    s = jnp.sum(ex, axis=1, keepdims=True)
    logp = lf - m - jnp.log(s)
    tgt = jnp.where(pseudo_ref[...] > 0, NUM_CLASSES, seg_ref[...])
    cols = lax.broadcasted_iota(jnp.int32, (N, NUM_CLASSES + 1), 1)
    nll = -jnp.sum(jnp.where(cols == tgt[:, None], logp, 0.0), axis=1)
    loss_ref[...] = jnp.full((1, 128), jnp.sum(nll) / N, jnp.float32)
    score_ref[...] = ex[:, NUM_CLASSES] / s[:, 0]


def _final_call(logits_full, segment, pseudo):
    return pl.pallas_call(
        _final_kernel,
        out_shape=(
            jax.ShapeDtypeStruct((N,), jnp.float32),
            jax.ShapeDtypeStruct((1, 128), jnp.float32),
        ),
    )(logits_full, segment, pseudo)


def kernel(coord, seg_logits, score, segment):
    msp, stop2 = _msp_call(seg_logits)
    dice = jax.random.randint(jax.random.key(1), (NUM_SEED,), 0,
                              int(0.2 * N)).astype(jnp.int32)
    dice2 = jnp.concatenate(
        [dice, jnp.full((64 - NUM_SEED,), -1, jnp.int32)]).reshape(1, 64)
    nbr, mask0 = _ball_call(coord, msp, dice2)
    params = jnp.zeros((16,), jnp.float32).at[0].set(stop2[0, 0])
    xs = jnp.asarray(coord[:, 0])
    ys = jnp.asarray(coord[:, 1])
    zs = jnp.asarray(coord[:, 2])
    pseudo = _sc_grow_call(xs, ys, zs, msp, nbr, mask0, params)
    logits_full = jnp.concatenate([seg_logits, score], axis=-1)
    score_out, loss2 = _final_call(logits_full, segment.astype(jnp.int32),
                                   pseudo)
    return (score_out, loss2[0, 0])


# speedup vs baseline: 1.8134x; 1.1833x over previous
import functools

import jax
import jax.numpy as jnp
from jax import lax
from jax.experimental import pallas as pl
from jax.experimental.pallas import tpu as pltpu
from jax.experimental.pallas import tpu_sc as plsc

N = 8192
NUM_CLASSES = 13
RADIUS2 = 0.25
KNN = 16
BETA = 0.5
NUM_SEED = 50
N_ITERS = 5
RBLK = 128
BIG = 1e30


def _msp_kernel(logits_ref, msp_ref, stop_ref):
    l = logits_ref[...]
    m = jnp.max(l, axis=1, keepdims=True)
    s = jnp.sum(jnp.exp(l - m), axis=1, keepdims=True)
    msp = 1.0 / s[:, 0]
    msp_ref[...] = msp
    mean = jnp.sum(msp) / N
    var = jnp.sum((msp - mean) * (msp - mean)) / N
    stop = mean - BETA * jnp.sqrt(jnp.maximum(var, 0.0))
    stop_ref[...] = jnp.full((1, 128), stop, jnp.float32)


def _msp_call(seg_logits):
    return pl.pallas_call(
        _msp_kernel,
        out_shape=(
            jax.ShapeDtypeStruct((N,), jnp.float32),
            jax.ShapeDtypeStruct((1, 128), jnp.float32),
        ),
    )(seg_logits)


def _ball_kernel(cblk_ref, ct_ref, mspf_ref, mspb_ref, dice_ref,
                 nbr_ref, mask0_ref):
    i = pl.program_id(0)
    cblk = cblk_ref[...]
    ct = ct_ref[...]
    sqf = jnp.sum(ct * ct, axis=0)[None, :]
    sqb = jnp.sum(cblk * cblk, axis=1)[:, None]
    d2 = sqb + sqf - 2.0 * jnp.dot(cblk, ct, preferred_element_type=jnp.float32)
    jl = lax.broadcasted_iota(jnp.int32, (RBLK, N), 1)
    d2v = jnp.where(d2 <= RADIUS2, d2, BIG)
    cols = []
    for _ in range(KNN):
        m = jnp.min(d2v, axis=1, keepdims=True)
        idx = jnp.min(jnp.where(d2v == m, jl, N), axis=1, keepdims=True)
        cols.append(jnp.where(m[:, 0] < 0.5 * BIG, idx[:, 0], -1))
        d2v = jnp.where(jl == idx, BIG, d2v)
    nbr_blk = jnp.stack(cols, axis=1).astype(jnp.int32)
    nbr_ref[...] = jnp.concatenate(
        [nbr_blk, jnp.full((RBLK, 128 - KNN), -1, jnp.int32)], axis=1)

    mspf = mspf_ref[0, :][None, :]
    mspb = mspb_ref[...][:, None]
    gi = i * RBLK + lax.broadcasted_iota(jnp.int32, (RBLK, 1), 0)
    rank = (jnp.sum((mspf < mspb).astype(jnp.int32), axis=1)
            + jnp.sum(((mspf == mspb) & (jl < gi)).astype(jnp.int32), axis=1))
    dice = dice_ref[...]
    hit = jnp.any(rank[:, None] == dice, axis=1)
    mask0_ref[...] = hit.astype(jnp.int32)


def _ball_call(coord, msp, dice2):
    coordT = coord.T
    msp2 = msp.reshape(1, N)
    grid = (N // RBLK,)
    return pl.pallas_call(
        _ball_kernel,
        grid=grid,
        in_specs=[
            pl.BlockSpec((RBLK, 3), lambda i: (i, 0)),
            pl.BlockSpec((3, N), lambda i: (0, 0)),
            pl.BlockSpec((1, N), lambda i: (0, 0)),
            pl.BlockSpec((RBLK,), lambda i: (i,)),
            pl.BlockSpec((1, 64), lambda i: (0, 0)),
        ],
        out_specs=(
            pl.BlockSpec((RBLK, 128), lambda i: (i, 0)),
            pl.BlockSpec((RBLK,), lambda i: (i,)),
        ),
        out_shape=(
            jax.ShapeDtypeStruct((N, 128), jnp.int32),
            jax.ShapeDtypeStruct((N,), jnp.int32),
        ),
    )(coord, coordT, msp2, msp, dice2)


NG = N // 16


def _sdiv(a, b):
    return (jnp.full((16,), a, jnp.float32) / jnp.full((16,), b, jnp.float32))[0]


def _vsqrt(a):
    a = jnp.maximum(a, 1e-30)
    i = plsc.bitcast(a, jnp.int32)
    y = plsc.bitcast(0x5F3759DF - (i >> 1), jnp.float32)
    for _ in range(3):
        y = y * (1.5 - 0.5 * a * y * y)
    return a * y


def _sc_grow_body(x_h, y_h, z_h, s_h, nbr_h, mask0_h, params_h, out_h,
                  x_v, y_v, z_v, s_v, d_v, sim_v, csim_v,
                  mask_v, cand_v, idxl_v, buf_v, sem):
    tile0 = (lax.axis_index("c") == 0) & (lax.axis_index("s") == 0)

    @pl.when(tile0)
    def _():
        pltpu.sync_copy(x_h, x_v)
        pltpu.sync_copy(y_h, y_v)
        pltpu.sync_copy(z_h, z_v)
        pltpu.sync_copy(s_h, s_v)
        pltpu.sync_copy(mask0_h, mask_v)
        pltpu.sync_copy(params_h, csim_v.at[pl.ds(0, 16)])
        stop_cond = csim_v[pl.ds(0, 16)][0]
        zeros_i = jnp.zeros((16,), jnp.int32)
        zeros_f = jnp.zeros((16,), jnp.float32)
        ones_i = jnp.ones((16,), jnp.int32)
        lane = lax.iota(jnp.int32, 16)

        def iteration(_, carry):
            def red_body(g, acc):
                cnt_a, sx_a, sy_a, sz_a, ss_a = acc
                mv = mask_v[pl.ds(g * 16, 16)] > 0
                f = jnp.where(mv, 1.0, 0.0)
                return (cnt_a + f,
                        sx_a + jnp.where(mv, x_v[pl.ds(g * 16, 16)], 0.0),
                        sy_a + jnp.where(mv, y_v[pl.ds(g * 16, 16)], 0.0),
                        sz_a + jnp.where(mv, z_v[pl.ds(g * 16, 16)], 0.0),
                        ss_a + jnp.where(mv, s_v[pl.ds(g * 16, 16)], 0.0))

            cnt_a, sx_a, sy_a, sz_a, ss_a = lax.fori_loop(
                0, NG, red_body, (zeros_f, zeros_f, zeros_f, zeros_f, zeros_f),
                unroll=4)
            cnt = jnp.sum(cnt_a)
            denom = jnp.maximum(cnt, 1.0)
            rden = _sdiv(1.0, denom)
            mx = jnp.sum(sx_a) * rden
            my = jnp.sum(sy_a) * rden
            mz = jnp.sum(sz_a) * rden
            ms = jnp.sum(ss_a) * rden
            cond_met = ((ms > stop_cond) & (cnt > 0.01 * N) & (cnt > 50.0))

            @pl.when(jnp.logical_not(cond_met))
            def _grow():
                def cz_body(g, nm):
                    cand_v[pl.ds(g * 16, 16)] = zeros_i
                    mv = mask_v[pl.ds(g * 16, 16)] > 0
                    mi = jnp.where(mv, 1, 0)
                    pos = nm + plsc.cumsum(mi) - 1
                    plsc.store_scatter(idxl_v, [pos], g * 16 + lane, mask=mv)
                    return nm + jnp.sum(mi)

                nm = lax.fori_loop(0, NG, cz_body, jnp.int32(0), unroll=4)
                for t in range(8):
                    idxl_v[pl.ds(nm + t * 16, 16)] = zeros_i

                def chunk_body(c, _):
                    pltpu.async_copy(
                        nbr_h.at[idxl_v.at[pl.ds(c * 128, 128)]], buf_v,
                        sem).wait()

                    def row_body(r, __):
                        tgt = buf_v[r, pl.ds(0, 16)]
                        valid = tgt >= 0
                        plsc.store_scatter(cand_v,
                                           [jnp.where(valid, tgt, 0)],
                                           ones_i, mask=valid)
                        return 0

                    rhi = jnp.minimum(128, nm - c * 128)
                    lax.fori_loop(0, rhi, row_body, 0)
                    return 0

                nchunks = (nm + 127) >> 7
                lax.fori_loop(0, nchunks, chunk_body, 0)

                def dm_body(g, acc):
                    dmin_a, dmax_a = acc
                    sl = pl.ds(g * 16, 16)
                    cv = (cand_v[sl] > 0) & (mask_v[sl] == 0)
                    cand_v[sl] = jnp.where(cv, 1, 0)
                    dx = x_v[sl] - mx
                    dy = y_v[sl] - my
                    dz = z_v[sl] - mz
                    d = _vsqrt(dx * dx + dy * dy + dz * dz)
                    d_v[sl] = d
                    return (jnp.minimum(dmin_a, jnp.where(cv, d, BIG)),
                            jnp.maximum(dmax_a, jnp.where(cv, d, -BIG)))

                dmin_a, dmax_a = lax.fori_loop(
                    0, NG, dm_body,
                    (jnp.full((16,), BIG, jnp.float32),
                     jnp.full((16,), -BIG, jnp.float32)), unroll=4)
                dmin = jnp.min(dmin_a)
                dmax = jnp.max(dmax_a)

                inv_rng = _sdiv(1.0, dmax - dmin + 0.001)

                def sim_body(g, nc):
                    sl = pl.ds(g * 16, 16)
                    cv = cand_v[sl] > 0
                    dist_sim = 1.0 - (d_v[sl] - dmin) * inv_rng
                    conf_sim = jnp.exp(-jnp.abs(s_v[sl] - ms))
                    sim = 0.4 * dist_sim + 0.6 * conf_sim
                    sim = jnp.where(cv, sim, 0.0)
                    sim_v[sl] = sim
                    ci = jnp.where(cv, 1, 0)
                    pos = nc + plsc.cumsum(ci) - 1
                    plsc.store_scatter(csim_v, [pos], sim, mask=cv)
                    return nc + jnp.sum(ci)

                nc = lax.fori_loop(0, NG, sim_body, jnp.int32(0), unroll=4)
                csim_v[pl.ds(nc, 16)] = zeros_f

                k_sel = (0.4 * nc.astype(jnp.float32)).astype(jnp.int32)
                ncg = (nc + 15) >> 4

                def bs_body(_, lohi):
                    lo, hi = lohi
                    mid = (lo + hi) >> 1
                    midf = lax.bitcast_convert_type(mid, jnp.float32)

                    def cb(g, a):
                        return a + jnp.where(csim_v[pl.ds(g * 16, 16)] >= midf,
                                             1, 0)

                    c = jnp.sum(lax.fori_loop(0, ncg, cb, zeros_i))
                    take = c >= k_sel
                    return (jnp.where(take, mid, lo), jnp.where(take, hi, mid))

                lo, _hi = lax.fori_loop(0, 31, bs_body,
                                        (jnp.int32(0), jnp.int32(0x3F800001)))
                thr = lax.bitcast_convert_type(lo, jnp.float32)
                do_sel = k_sel > 0

                def sel_body(g, _):
                    sl = pl.ds(g * 16, 16)
                    sel = (cand_v[sl] > 0) & (sim_v[sl] >= thr) & do_sel
                    mask_v[sl] = jnp.where(sel, 1, mask_v[sl])
                    return 0

                lax.fori_loop(0, NG, sel_body, 0, unroll=4)

            return carry

        lax.fori_loop(0, N_ITERS, iteration, 0)
        pltpu.sync_copy(mask_v, out_h)


def _sc_grow_call(xs, ys, zs, ss, nbr, mask0, params):
    mesh = plsc.VectorSubcoreMesh(core_axis_name="c", subcore_axis_name="s",
                                  num_cores=2, num_subcores=16)
    f32 = jnp.float32
    i32 = jnp.int32
    kern = pl.kernel(
        _sc_grow_body,
        out_type=jax.ShapeDtypeStruct((N,), i32),
        mesh=mesh,
        scratch_types=[
            pltpu.VMEM((N,), f32),
            pltpu.VMEM((N,), f32),
            pltpu.VMEM((N,), f32),
            pltpu.VMEM((N,), f32),
            pltpu.VMEM((N,), f32),
            pltpu.VMEM((N,), f32),
            pltpu.VMEM((N + 16,), f32),
            pltpu.VMEM((N,), i32),
            pltpu.VMEM((N,), i32),
            pltpu.VMEM((N + 128,), i32),
            pltpu.VMEM((128, 128), i32),
            pltpu.SemaphoreType.DMA,
        ],
        compiler_params=pltpu.CompilerParams(needs_layout_passes=False),
    )
    return kern(xs, ys, zs, ss, nbr, mask0, params)


def _final_kernel(lf_ref, seg_ref, pseudo_ref, score_ref, loss_ref):
    lf = lf_ref[...]
    m = jnp.max(lf, axis=1, keepdims=True)
    ex = jnp.exp(lf - m)
    s = jnp.sum(ex, axis=1, keepdims=True)
    logp = lf - m - jnp.log(s)
    tgt = jnp.where(pseudo_ref[...] > 0, NUM_CLASSES, seg_ref[...])
    cols = lax.broadcasted_iota(jnp.int32, (N, NUM_CLASSES + 1), 1)
    nll = -jnp.sum(jnp.where(cols == tgt[:, None], logp, 0.0), axis=1)
    loss_ref[...] = jnp.full((1, 128), jnp.sum(nll) / N, jnp.float32)
    score_ref[...] = ex[:, NUM_CLASSES] / s[:, 0]


def _final_call(logits_full, segment, pseudo):
    return pl.pallas_call(
        _final_kernel,
        out_shape=(
            jax.ShapeDtypeStruct((N,), jnp.float32),
            jax.ShapeDtypeStruct((1, 128), jnp.float32),
        ),
    )(logits_full, segment, pseudo)


def kernel(coord, seg_logits, score, segment):
    msp, stop2 = _msp_call(seg_logits)
    dice = jax.random.randint(jax.random.key(1), (NUM_SEED,), 0,
                              int(0.2 * N)).astype(jnp.int32)
    dice2 = jnp.concatenate(
        [dice, jnp.full((64 - NUM_SEED,), -1, jnp.int32)]).reshape(1, 64)
    nbr, mask0 = _ball_call(coord, msp, dice2)
    coordT = coord.T
    pseudo = _sc_grow_call(coordT[0], coordT[1], coordT[2], msp, nbr, mask0,
                           stop2.reshape(128)[:16])
    logits_full = jnp.concatenate([seg_logits, score], axis=-1)
    score_out, loss2 = _final_call(logits_full, segment.astype(jnp.int32),
                                   pseudo)
    return (score_out, loss2[0, 0])
